# Initial kernel scaffold; baseline (speedup 1.0000x reference)
import functools

import jax
import jax.numpy as jnp
from jax import lax
from jax.experimental import pallas as pl
from jax.experimental.pallas import tpu as pltpu
from jax.experimental.pallas import tpu_sc as plsc

NNODE = 10000
NPAD = 10240
NEDGE = 320000
CH = 128
CPT = 80
NTILES = 32
EP = CH * CPT * NTILES
RPT = NPAD // 16
HW = 8
BLK = 1280
GRID = NPAD // BLK
F_IN = 128
HID = 64
C = 40

_MESH = plsc.VectorSubcoreMesh(core_axis_name="c", subcore_axis_name="s")


def _f32(shape):
    return jax.ShapeDtypeStruct(shape, jnp.float32)


@functools.partial(
    pl.kernel,
    out_type=(_f32((2, NPAD, HW)), _f32((2, NPAD, HW)), _f32((2, NPAD, HW))),
    mesh=_MESH,
    scratch_types=[
        pltpu.VMEM((CPT, CH), jnp.int32),
        pltpu.VMEM((CH, HW), jnp.float32),
        pltpu.VMEM_SHARED((NPAD, HW), jnp.float32),
        pltpu.VMEM_SHARED((NPAD, HW), jnp.float32),
        pltpu.VMEM_SHARED((NPAD, HW), jnp.float32),
    ],
)
def _sc_hist(col_h, nid_h, hid_h, ones_h, zeros_h, deg_o, d_o, b_o,
             idx_v, ones_v, acc0, acc1, acc2):
    core = lax.axis_index("c")
    sid = lax.axis_index("s")
    wid = core * 16 + sid
    r0 = sid * RPT
    pltpu.sync_copy(ones_h, ones_v)
    pltpu.sync_copy(zeros_h.at[pl.ds(r0, RPT)], acc0.at[pl.ds(r0, RPT)])
    pltpu.sync_copy(zeros_h.at[pl.ds(r0, RPT)], acc1.at[pl.ds(r0, RPT)])
    pltpu.sync_copy(zeros_h.at[pl.ds(r0, RPT)], acc2.at[pl.ds(r0, RPT)])
    plsc.subcore_barrier()
    for ih, acc in ((col_h, acc0), (nid_h, acc1), (hid_h, acc2)):
        pltpu.sync_copy(ih.at[pl.ds(wid * CPT, CPT)], idx_v)

        def body(c, carry, acc=acc):
            pltpu.sync_copy(ones_v, acc.at[idx_v.at[c]], add=True)
            return carry

        lax.fori_loop(0, CPT, body, 0)
    plsc.subcore_barrier()
    for acc, out in ((acc0, deg_o), (acc1, d_o), (acc2, b_o)):
        pltpu.sync_copy(acc.at[pl.ds(r0, RPT)], out.at[core, pl.ds(r0, RPT)])


def _edge_pass(src_h, dst_h, tbl_h, acc, is_v, id_v, rb0, rb1, sem0, sem1, wid):
    pltpu.sync_copy(src_h.at[pl.ds(wid * CPT, CPT)], is_v)
    pltpu.sync_copy(dst_h.at[pl.ds(wid * CPT, CPT)], id_v)
    pltpu.async_copy(tbl_h.at[is_v.at[0]], rb0, sem0)
    pltpu.async_copy(tbl_h.at[is_v.at[1]], rb1, sem1)
    npair = CPT // 2

    def body(p, carry):
        c0 = 2 * p
        pltpu.make_async_copy(tbl_h.at[is_v.at[c0]], rb0, sem0).wait()
        pltpu.sync_copy(rb0, acc.at[id_v.at[c0]], add=True)

        @pl.when(p + 1 < npair)
        def _():
            pltpu.async_copy(tbl_h.at[is_v.at[c0 + 2]], rb0, sem0)

        pltpu.make_async_copy(tbl_h.at[is_v.at[c0 + 1]], rb1, sem1).wait()
        pltpu.sync_copy(rb1, acc.at[id_v.at[c0 + 1]], add=True)

        @pl.when(p + 1 < npair)
        def _():
            pltpu.async_copy(tbl_h.at[is_v.at[c0 + 3]], rb1, sem1)

        return carry

    lax.fori_loop(0, npair, body, 0)


def _make_sc_scatter(fs):
    n = len(fs)
    scratch = []
    for f in fs:
        scratch += [
            pltpu.VMEM((CPT, CH), jnp.int32),
            pltpu.VMEM((CPT, CH), jnp.int32),
            pltpu.VMEM((CH, f), jnp.float32),
            pltpu.VMEM((CH, f), jnp.float32),
            pltpu.VMEM_SHARED((NPAD, f), jnp.float32),
        ]
    scratch += [pltpu.SemaphoreType.DMA, pltpu.SemaphoreType.DMA]

    def body(*refs):
        ins = refs[: 4 * n]
        outs = refs[4 * n: 5 * n]
        scr = refs[5 * n:]
        sem0, sem1 = scr[-2], scr[-1]
        core = lax.axis_index("c")
        sid = lax.axis_index("s")
        wid = core * 16 + sid
        r0 = sid * RPT
        for i in range(n):
            zeros_h = ins[4 * i + 3]
            acc = scr[5 * i + 4]
            pltpu.sync_copy(zeros_h.at[pl.ds(r0, RPT)], acc.at[pl.ds(r0, RPT)])
        plsc.subcore_barrier()
        for i in range(n):
            src_h, dst_h, tbl_h, _ = ins[4 * i: 4 * i + 4]
            is_v, id_v, rb0, rb1, acc = scr[5 * i: 5 * i + 5]
            _edge_pass(src_h, dst_h, tbl_h, acc, is_v, id_v, rb0, rb1,
                       sem0, sem1, wid)
            plsc.subcore_barrier()
            pltpu.sync_copy(acc.at[pl.ds(r0, RPT)],
                            outs[i].at[core, pl.ds(r0, RPT)])

    return pl.kernel(
        body,
        out_type=tuple(_f32((2, NPAD, f)) for f in fs),
        mesh=_MESH,
        scratch_types=scratch,
    )


_sc_pass_64_64 = _make_sc_scatter((HID, HID))
_sc_pass_64_40 = _make_sc_scatter((HID, C))
_sc_pass_40 = _make_sc_scatter((C,))


def _row_spec(f):
    return pl.BlockSpec((BLK, f), lambda i: (i, 0))


def _part_spec(f):
    return pl.BlockSpec((2, BLK, f), lambda i: (0, i, 0))


def _full_spec(shape):
    return pl.BlockSpec(shape, lambda i: tuple(0 for _ in shape))


def _tc1_body(x_ref, wg_ref, wh_ref, deg_ref, d_ref, b_ref,
              y1_ref, xh1_ref, dinv_ref, dinvh_ref, binv_ref):
    x = x_ref[...]
    deg = deg_ref[0, :, 0:1] + deg_ref[1, :, 0:1] + 1.0
    dinv = lax.rsqrt(deg)
    dd = d_ref[0, :, 0:1] + d_ref[1, :, 0:1]
    bb = b_ref[0, :, 0:1] + b_ref[1, :, 0:1]
    y1_ref[...] = dinv * jnp.dot(x, wg_ref[...], preferred_element_type=jnp.float32)
    xh1_ref[...] = jnp.dot(x, wh_ref[...], preferred_element_type=jnp.float32)
    dinv_ref[...] = dinv
    dinvh_ref[...] = jnp.where(dd > 0, 1.0 / dd, 0.0)
    binv_ref[...] = jnp.where(bb > 0, 1.0 / bb, 0.0)


def _tc1(x_pad, wg1, wh1, deg_p, d_p, b_p):
    return pl.pallas_call(
        _tc1_body,
        grid=(GRID,),
        in_specs=[_row_spec(F_IN), _full_spec((F_IN, HID)), _full_spec((F_IN, HID)),
                  _part_spec(HW), _part_spec(HW), _part_spec(HW)],
        out_specs=[_row_spec(HID), _row_spec(HID), _row_spec(1), _row_spec(1), _row_spec(1)],
        out_shape=[_f32((NPAD, HID)), _f32((NPAD, HID)), _f32((NPAD, 1)),
                   _f32((NPAD, 1)), _f32((NPAD, 1))],
    )(x_pad, wg1, wh1, deg_p, d_p, b_p)


def _tc2_body(s1_ref, t1_ref, y1_ref, dinv_ref, binv_ref, bg1_ref, wg2_ref,
              y2_ref, ef1_ref):
    dinv = dinv_ref[...]
    xg1 = jnp.maximum(dinv * (s1_ref[0] + s1_ref[1] + y1_ref[...]) + bg1_ref[...], 0.0)
    y2_ref[...] = dinv * jnp.dot(xg1, wg2_ref[...], preferred_element_type=jnp.float32)
    ef1_ref[...] = binv_ref[...] * (t1_ref[0] + t1_ref[1])


def _tc2(s1_p, t1_p, y1, dinv, binv, bg1, wg2):
    return pl.pallas_call(
        _tc2_body,
        grid=(GRID,),
        in_specs=[_part_spec(HID), _part_spec(HID), _row_spec(HID), _row_spec(1),
                  _row_spec(1), _full_spec((1, HID)), _full_spec((HID, C))],
        out_specs=[_row_spec(C), _row_spec(HID)],
        out_shape=[_f32((NPAD, C)), _f32((NPAD, HID))],
    )(s1_p, t1_p, y1, dinv, binv, bg1, wg2)


def _tc3_body(u1_ref, s2_ref, y2_ref, dinv_ref, dinvh_ref, bh1_ref, wh2_ref,
              bg2_ref, xh2_ref, xg2_ref):
    xhyp1 = jnp.maximum(dinvh_ref[...] * (u1_ref[0] + u1_ref[1]) + bh1_ref[...], 0.0)
    xh2_ref[...] = jnp.dot(xhyp1, wh2_ref[...], preferred_element_type=jnp.float32)
    xg2_ref[...] = dinv_ref[...] * (s2_ref[0] + s2_ref[1] + y2_ref[...]) + bg2_ref[...]


def _tc3(u1_p, s2_p, y2, dinv, dinvh, bh1, wh2, bg2):
    return pl.pallas_call(
        _tc3_body,
        grid=(GRID,),
        in_specs=[_part_spec(HID), _part_spec(C), _row_spec(C), _row_spec(1),
                  _row_spec(1), _full_spec((1, HID)), _full_spec((HID, C)),
                  _full_spec((1, C))],
        out_specs=[_row_spec(C), _row_spec(C)],
        out_shape=[_f32((NPAD, C)), _f32((NPAD, C))],
    )(u1_p, s2_p, y2, dinv, dinvh, bh1, wh2, bg2)


def _tc4_body(t2_ref, binv_ref, ef2_ref):
    ef2_ref[...] = binv_ref[...] * (t2_ref[0] + t2_ref[1])


def _tc4(t2_p, binv):
    return pl.pallas_call(
        _tc4_body,
        grid=(GRID,),
        in_specs=[_part_spec(C), _row_spec(1)],
        out_specs=[_row_spec(C)],
        out_shape=[_f32((NPAD, C))],
    )(t2_p, binv)[0]


def _tc5_body(u2_ref, dinvh_ref, bh2_ref, xg2_ref, wlp_ref, blp_ref, out_ref):
    xhyp2 = dinvh_ref[...] * (u2_ref[0] + u2_ref[1]) + bh2_ref[...]
    cat = jnp.concatenate([xg2_ref[...], xhyp2], axis=1)
    o = jnp.dot(cat, wlp_ref[...], preferred_element_type=jnp.float32) + blp_ref[...]
    m = jnp.max(o, axis=1, keepdims=True)
    s = o - m
    out_ref[...] = s - jnp.log(jnp.sum(jnp.exp(s), axis=1, keepdims=True))


def _tc5(u2_p, dinvh, bh2, xg2, wlp, blp):
    return pl.pallas_call(
        _tc5_body,
        grid=(GRID,),
        in_specs=[_part_spec(C), _row_spec(1), _full_spec((1, C)), _row_spec(C),
                  _full_spec((2 * C, C)), _full_spec((1, C))],
        out_specs=[_row_spec(C)],
        out_shape=[_f32((NPAD, C))],
    )(u2_p, dinvh, bh2, xg2, wlp, blp)[0]


def kernel(x, edge_index, hyperedge_index, W_gcn1, b_gcn1, W_hyp1, b_hyp1,
           W_gcn2, b_gcn2, W_hyp2, b_hyp2, W_lp, b_lp):
    f32 = jnp.float32
    pad = jnp.full((EP - NEDGE,), NNODE, jnp.int32)

    def prep(a):
        return jnp.concatenate([a.astype(jnp.int32), pad]).reshape(NTILES * CPT, CH)

    row_p = prep(edge_index[0])
    col_p = prep(edge_index[1])
    nid_p = prep(hyperedge_index[0])
    hid_p = prep(hyperedge_index[1])
    x_pad = jnp.zeros((NPAD, F_IN), f32).at[:NNODE].set(x)
    ones8 = jnp.ones((CH, HW), f32)
    zeros8 = jnp.zeros((NPAD, HW), f32)
    z64 = jnp.zeros((NPAD, HID), f32)
    z40 = jnp.zeros((NPAD, C), f32)
    bg1 = b_gcn1.reshape(1, HID)
    bh1 = b_hyp1.reshape(1, HID)
    bg2 = b_gcn2.reshape(1, C)
    bh2 = b_hyp2.reshape(1, C)
    blp = b_lp.reshape(1, C)

    deg_p, d_p, b_p = _sc_hist(col_p, nid_p, hid_p, ones8, zeros8)
    y1, xh1, dinv, dinvh, binv = _tc1(x_pad, W_gcn1, W_hyp1, deg_p, d_p, b_p)
    s1_p, t1_p = _sc_pass_64_64(row_p, col_p, y1, z64, nid_p, hid_p, xh1, z64)
    y2, ef1 = _tc2(s1_p, t1_p, y1, dinv, binv, bg1, W_gcn2)
    u1_p, s2_p = _sc_pass_64_40(hid_p, nid_p, ef1, z64, row_p, col_p, y2, z40)
    xh2, xg2 = _tc3(u1_p, s2_p, y2, dinv, dinvh, bh1, W_hyp2, bg2)
    (t2_p,) = _sc_pass_40(nid_p, hid_p, xh2, z40)
    ef2 = _tc4(t2_p, binv)
    (u2_p,) = _sc_pass_40(hid_p, nid_p, ef2, z40)
    out = _tc5(u2_p, dinvh, bh2, xg2, W_lp, blp)
    return out[:NNODE]


# speedup vs baseline: 11.6133x; 11.6133x over previous
import functools

import jax
import jax.numpy as jnp
from jax import lax
from jax.experimental import pallas as pl
from jax.experimental.pallas import tpu as pltpu
from jax.experimental.pallas import tpu_sc as plsc

NNODE = 10000
NPAD = 10240
NEDGE = 320000
CH = 128
CPT = 80
NTILES = 32
EP = CH * CPT * NTILES
RPT = NPAD // 16
HW = 8
BLK = 1280
GRID = NPAD // BLK
F_IN = 128
HID = 64
C = 40

_MESH = plsc.VectorSubcoreMesh(core_axis_name="c", subcore_axis_name="s")


def _f32(shape):
    return jax.ShapeDtypeStruct(shape, jnp.float32)


@functools.partial(
    pl.kernel,
    out_type=(_f32((2, NPAD, HW)), _f32((2, NPAD, HW)), _f32((2, NPAD, HW))),
    mesh=_MESH,
    scratch_types=[
        pltpu.VMEM((CPT, CH), jnp.int32),
        pltpu.VMEM((CH, HW), jnp.float32),
        pltpu.VMEM_SHARED((NPAD, HW), jnp.float32),
        pltpu.VMEM_SHARED((NPAD, HW), jnp.float32),
        pltpu.VMEM_SHARED((NPAD, HW), jnp.float32),
    ],
    compiler_params=pltpu.CompilerParams(use_tc_tiling_on_sc=False),
)
def _sc_hist(col_h, nid_h, hid_h, ones_h, zeros_h, deg_o, d_o, b_o,
             idx_v, ones_v, acc0, acc1, acc2):
    core = lax.axis_index("c")
    sid = lax.axis_index("s")
    wid = core * 16 + sid
    r0 = sid * RPT
    pltpu.sync_copy(ones_h, ones_v)
    pltpu.sync_copy(zeros_h.at[pl.ds(r0, RPT)], acc0.at[pl.ds(r0, RPT)])
    pltpu.sync_copy(zeros_h.at[pl.ds(r0, RPT)], acc1.at[pl.ds(r0, RPT)])
    pltpu.sync_copy(zeros_h.at[pl.ds(r0, RPT)], acc2.at[pl.ds(r0, RPT)])
    plsc.subcore_barrier()
    for ih, acc in ((col_h, acc0), (nid_h, acc1), (hid_h, acc2)):
        pltpu.sync_copy(ih.at[pl.ds(wid * CPT, CPT)], idx_v)

        def body(c, carry, acc=acc):
            pltpu.sync_copy(ones_v, acc.at[idx_v.at[c]], add=True)
            return carry

        lax.fori_loop(0, CPT, body, 0)
    plsc.subcore_barrier()
    for acc, out in ((acc0, deg_o), (acc1, d_o), (acc2, b_o)):
        pltpu.sync_copy(acc.at[pl.ds(r0, RPT)], out.at[core, pl.ds(r0, RPT)])


def _edge_pass(src_h, dst_h, tbl_h, acc, is_v, id_v, rb0, rb1, sem0, sem1, wid):
    pltpu.sync_copy(src_h.at[pl.ds(wid * CPT, CPT)], is_v)
    pltpu.sync_copy(dst_h.at[pl.ds(wid * CPT, CPT)], id_v)
    pltpu.async_copy(tbl_h.at[is_v.at[0]], rb0, sem0)
    pltpu.async_copy(tbl_h.at[is_v.at[1]], rb1, sem1)
    npair = CPT // 2

    def body(p, carry):
        c0 = 2 * p
        pltpu.make_async_copy(tbl_h.at[is_v.at[c0]], rb0, sem0).wait()
        pltpu.sync_copy(rb0, acc.at[id_v.at[c0]], add=True)

        @pl.when(p + 1 < npair)
        def _():
            pltpu.async_copy(tbl_h.at[is_v.at[c0 + 2]], rb0, sem0)

        pltpu.make_async_copy(tbl_h.at[is_v.at[c0 + 1]], rb1, sem1).wait()
        pltpu.sync_copy(rb1, acc.at[id_v.at[c0 + 1]], add=True)

        @pl.when(p + 1 < npair)
        def _():
            pltpu.async_copy(tbl_h.at[is_v.at[c0 + 3]], rb1, sem1)

        return carry

    lax.fori_loop(0, npair, body, 0)


def _make_sc_scatter(fs):
    n = len(fs)
    widths = sorted(set(fs))
    scratch = []
    for f in fs:
        scratch += [
            pltpu.VMEM((CPT, CH), jnp.int32),
            pltpu.VMEM((CPT, CH), jnp.int32),
            pltpu.VMEM((CH, f), jnp.float32),
            pltpu.VMEM((CH, f), jnp.float32),
        ]
    for f in widths:
        scratch.append(pltpu.VMEM_SHARED((NPAD, f), jnp.float32))
    scratch += [pltpu.SemaphoreType.DMA, pltpu.SemaphoreType.DMA]

    def body(*refs):
        ins = refs[: 4 * n]
        outs = refs[4 * n: 5 * n]
        scr = refs[5 * n:]
        sem0, sem1 = scr[-2], scr[-1]
        accs = {f: scr[4 * n + j] for j, f in enumerate(widths)}
        core = lax.axis_index("c")
        sid = lax.axis_index("s")
        wid = core * 16 + sid
        r0 = sid * RPT
        seen = set()
        for i in range(n):
            f = fs[i]
            if f in seen:
                continue
            seen.add(f)
            zeros_h = ins[4 * i + 3]
            pltpu.sync_copy(zeros_h.at[pl.ds(r0, RPT)],
                            accs[f].at[pl.ds(r0, RPT)])
        plsc.subcore_barrier()
        for i in range(n):
            src_h, dst_h, tbl_h, zeros_h = ins[4 * i: 4 * i + 4]
            is_v, id_v, rb0, rb1 = scr[4 * i: 4 * i + 4]
            acc = accs[fs[i]]
            _edge_pass(src_h, dst_h, tbl_h, acc, is_v, id_v, rb0, rb1,
                       sem0, sem1, wid)
            plsc.subcore_barrier()
            pltpu.sync_copy(acc.at[pl.ds(r0, RPT)],
                            outs[i].at[core, pl.ds(r0, RPT)])
            if fs[i] in [fj for fj in fs[i + 1:]]:
                pltpu.sync_copy(zeros_h.at[pl.ds(r0, RPT)],
                                acc.at[pl.ds(r0, RPT)])
            if i + 1 < n:
                plsc.subcore_barrier()

    return pl.kernel(
        body,
        out_type=tuple(_f32((2, NPAD, f)) for f in fs),
        mesh=_MESH,
        scratch_types=scratch,
        compiler_params=pltpu.CompilerParams(use_tc_tiling_on_sc=False),
    )


_sc_pass_64 = _make_sc_scatter((HID,))
_sc_pass_40 = _make_sc_scatter((C,))


def _row_spec(f):
    return pl.BlockSpec((BLK, f), lambda i: (i, 0))


def _part_spec(f):
    return pl.BlockSpec((2, BLK, f), lambda i: (0, i, 0))


def _full_spec(shape):
    return pl.BlockSpec(shape, lambda i: tuple(0 for _ in shape))


def _tc1_body(x_ref, wg_ref, wh_ref, deg_ref, d_ref, b_ref,
              y1_ref, xh1_ref, dinv_ref, dinvh_ref, binv_ref):
    x = x_ref[...]
    deg = deg_ref[0, :, 0:1] + deg_ref[1, :, 0:1] + 1.0
    dinv = lax.rsqrt(deg)
    dd = d_ref[0, :, 0:1] + d_ref[1, :, 0:1]
    bb = b_ref[0, :, 0:1] + b_ref[1, :, 0:1]
    y1_ref[...] = dinv * jnp.dot(x, wg_ref[...], preferred_element_type=jnp.float32)
    xh1_ref[...] = jnp.dot(x, wh_ref[...], preferred_element_type=jnp.float32)
    dinv_ref[...] = dinv
    dinvh_ref[...] = jnp.where(dd > 0, 1.0 / dd, 0.0)
    binv_ref[...] = jnp.where(bb > 0, 1.0 / bb, 0.0)


def _tc1(x_pad, wg1, wh1, deg_p, d_p, b_p):
    return pl.pallas_call(
        _tc1_body,
        grid=(GRID,),
        in_specs=[_row_spec(F_IN), _full_spec((F_IN, HID)), _full_spec((F_IN, HID)),
                  _part_spec(HW), _part_spec(HW), _part_spec(HW)],
        out_specs=[_row_spec(HID), _row_spec(HID), _row_spec(1), _row_spec(1), _row_spec(1)],
        out_shape=[_f32((NPAD, HID)), _f32((NPAD, HID)), _f32((NPAD, 1)),
                   _f32((NPAD, 1)), _f32((NPAD, 1))],
    )(x_pad, wg1, wh1, deg_p, d_p, b_p)


def _tc2_body(s1_ref, t1_ref, y1_ref, dinv_ref, binv_ref, bg1_ref, wg2_ref,
              y2_ref, ef1_ref):
    dinv = dinv_ref[...]
    xg1 = jnp.maximum(dinv * (s1_ref[0] + s1_ref[1] + y1_ref[...]) + bg1_ref[...], 0.0)
    y2_ref[...] = dinv * jnp.dot(xg1, wg2_ref[...], preferred_element_type=jnp.float32)
    ef1_ref[...] = binv_ref[...] * (t1_ref[0] + t1_ref[1])


def _tc2(s1_p, t1_p, y1, dinv, binv, bg1, wg2):
    return pl.pallas_call(
        _tc2_body,
        grid=(GRID,),
        in_specs=[_part_spec(HID), _part_spec(HID), _row_spec(HID), _row_spec(1),
                  _row_spec(1), _full_spec((1, HID)), _full_spec((HID, C))],
        out_specs=[_row_spec(C), _row_spec(HID)],
        out_shape=[_f32((NPAD, C)), _f32((NPAD, HID))],
    )(s1_p, t1_p, y1, dinv, binv, bg1, wg2)


def _tc3_body(u1_ref, s2_ref, y2_ref, dinv_ref, dinvh_ref, bh1_ref, wh2_ref,
              bg2_ref, xh2_ref, xg2_ref):
    xhyp1 = jnp.maximum(dinvh_ref[...] * (u1_ref[0] + u1_ref[1]) + bh1_ref[...], 0.0)
    xh2_ref[...] = jnp.dot(xhyp1, wh2_ref[...], preferred_element_type=jnp.float32)
    xg2_ref[...] = dinv_ref[...] * (s2_ref[0] + s2_ref[1] + y2_ref[...]) + bg2_ref[...]


def _tc3(u1_p, s2_p, y2, dinv, dinvh, bh1, wh2, bg2):
    return pl.pallas_call(
        _tc3_body,
        grid=(GRID,),
        in_specs=[_part_spec(HID), _part_spec(C), _row_spec(C), _row_spec(1),
                  _row_spec(1), _full_spec((1, HID)), _full_spec((HID, C)),
                  _full_spec((1, C))],
        out_specs=[_row_spec(C), _row_spec(C)],
        out_shape=[_f32((NPAD, C)), _f32((NPAD, C))],
    )(u1_p, s2_p, y2, dinv, dinvh, bh1, wh2, bg2)


def _tc4_body(t2_ref, binv_ref, ef2_ref):
    ef2_ref[...] = binv_ref[...] * (t2_ref[0] + t2_ref[1])


def _tc4(t2_p, binv):
    return pl.pallas_call(
        _tc4_body,
        grid=(GRID,),
        in_specs=[_part_spec(C), _row_spec(1)],
        out_specs=[_row_spec(C)],
        out_shape=[_f32((NPAD, C))],
    )(t2_p, binv)[0]


def _tc5_body(u2_ref, dinvh_ref, bh2_ref, xg2_ref, wlp_ref, blp_ref, out_ref):
    xhyp2 = dinvh_ref[...] * (u2_ref[0] + u2_ref[1]) + bh2_ref[...]
    cat = jnp.concatenate([xg2_ref[...], xhyp2], axis=1)
    o = jnp.dot(cat, wlp_ref[...], preferred_element_type=jnp.float32) + blp_ref[...]
    m = jnp.max(o, axis=1, keepdims=True)
    s = o - m
    out_ref[...] = s - jnp.log(jnp.sum(jnp.exp(s), axis=1, keepdims=True))


def _tc5(u2_p, dinvh, bh2, xg2, wlp, blp):
    return pl.pallas_call(
        _tc5_body,
        grid=(GRID,),
        in_specs=[_part_spec(C), _row_spec(1), _full_spec((1, C)), _row_spec(C),
                  _full_spec((2 * C, C)), _full_spec((1, C))],
        out_specs=[_row_spec(C)],
        out_shape=[_f32((NPAD, C))],
    )(u2_p, dinvh, bh2, xg2, wlp, blp)[0]


def kernel(x, edge_index, hyperedge_index, W_gcn1, b_gcn1, W_hyp1, b_hyp1,
           W_gcn2, b_gcn2, W_hyp2, b_hyp2, W_lp, b_lp):
    f32 = jnp.float32
    pad = jnp.full((EP - NEDGE,), NNODE, jnp.int32)

    def prep(a):
        return jnp.concatenate([a.astype(jnp.int32), pad]).reshape(NTILES * CPT, CH)

    row_p = prep(edge_index[0])
    col_p = prep(edge_index[1])
    nid_p = prep(hyperedge_index[0])
    hid_p = prep(hyperedge_index[1])
    x_pad = jnp.zeros((NPAD, F_IN), f32).at[:NNODE].set(x)
    ones8 = jnp.ones((CH, HW), f32)
    zeros8 = jnp.zeros((NPAD, HW), f32)
    z64 = jnp.zeros((NPAD, HID), f32)
    z40 = jnp.zeros((NPAD, C), f32)
    bg1 = b_gcn1.reshape(1, HID)
    bh1 = b_hyp1.reshape(1, HID)
    bg2 = b_gcn2.reshape(1, C)
    bh2 = b_hyp2.reshape(1, C)
    blp = b_lp.reshape(1, C)

    deg_p, d_p, b_p = _sc_hist(col_p, nid_p, hid_p, ones8, zeros8)
    y1, xh1, dinv, dinvh, binv = _tc1(x_pad, W_gcn1, W_hyp1, deg_p, d_p, b_p)
    (s1_p,) = _sc_pass_64(row_p, col_p, y1, z64)
    (t1_p,) = _sc_pass_64(nid_p, hid_p, xh1, z64)
    y2, ef1 = _tc2(s1_p, t1_p, y1, dinv, binv, bg1, W_gcn2)
    (u1_p,) = _sc_pass_64(hid_p, nid_p, ef1, z64)
    (s2_p,) = _sc_pass_40(row_p, col_p, y2, z40)
    xh2, xg2 = _tc3(u1_p, s2_p, y2, dinv, dinvh, bh1, W_hyp2, bg2)
    (t2_p,) = _sc_pass_40(nid_p, hid_p, xh2, z40)
    ef2 = _tc4(t2_p, binv)
    (u2_p,) = _sc_pass_40(hid_p, nid_p, ef2, z40)
    out = _tc5(u2_p, dinvh, bh2, xg2, W_lp, blp)
    return out[:NNODE]


# speedup vs baseline: 28.3968x; 2.4452x over previous
import functools

import jax
import jax.numpy as jnp
from jax import lax
from jax.experimental import pallas as pl
from jax.experimental.pallas import tpu as pltpu
from jax.experimental.pallas import tpu_sc as plsc

NNODE = 10000
NPAD = 10240
NEDGE = 320000
CH = 128
CPT = 80
NTILES = 32
EP = CH * CPT * NTILES
RPT = NPAD // 16
HW = 8
BLK = 1280
GRID = NPAD // BLK
F_IN = 128
HID = 64
C = 40

_MESH = plsc.VectorSubcoreMesh(core_axis_name="c", subcore_axis_name="s")


def _f32(shape):
    return jax.ShapeDtypeStruct(shape, jnp.float32)


@functools.partial(
    pl.kernel,
    out_type=(_f32((2, NPAD, HW)), _f32((2, NPAD, HW)), _f32((2, NPAD, HW))),
    mesh=_MESH,
    scratch_types=[
        pltpu.VMEM((CPT, CH), jnp.int32),
        pltpu.VMEM((CH, HW), jnp.float32),
        pltpu.VMEM_SHARED((NPAD, HW), jnp.float32),
        pltpu.VMEM_SHARED((NPAD, HW), jnp.float32),
        pltpu.VMEM_SHARED((NPAD, HW), jnp.float32),
    ],
    compiler_params=pltpu.CompilerParams(use_tc_tiling_on_sc=False),
)
def _sc_hist(col_h, nid_h, hid_h, ones_h, zeros_h, deg_o, d_o, b_o,
             idx_v, ones_v, acc0, acc1, acc2):
    core = lax.axis_index("c")
    sid = lax.axis_index("s")
    wid = core * 16 + sid
    r0 = sid * RPT
    pltpu.sync_copy(ones_h, ones_v)
    pltpu.sync_copy(zeros_h.at[pl.ds(r0, RPT)], acc0.at[pl.ds(r0, RPT)])
    pltpu.sync_copy(zeros_h.at[pl.ds(r0, RPT)], acc1.at[pl.ds(r0, RPT)])
    pltpu.sync_copy(zeros_h.at[pl.ds(r0, RPT)], acc2.at[pl.ds(r0, RPT)])
    plsc.subcore_barrier()
    for ih, acc in ((col_h, acc0), (nid_h, acc1), (hid_h, acc2)):
        pltpu.sync_copy(ih.at[pl.ds(wid * CPT, CPT)], idx_v)

        def body(c, carry, acc=acc):
            pltpu.sync_copy(ones_v, acc.at[idx_v.at[c]], add=True)
            return carry

        lax.fori_loop(0, CPT, body, 0)
    plsc.subcore_barrier()
    for acc, out in ((acc0, deg_o), (acc1, d_o), (acc2, b_o)):
        pltpu.sync_copy(acc.at[pl.ds(r0, RPT)], out.at[core, pl.ds(r0, RPT)])


def _edge_pass(src_h, dst_h, tbl_h, acc, is_v, id_v, rb0, rb1, sem0, sem1, wid):
    pltpu.sync_copy(src_h.at[pl.ds(wid * CPT, CPT)], is_v)
    pltpu.sync_copy(dst_h.at[pl.ds(wid * CPT, CPT)], id_v)
    pltpu.async_copy(tbl_h.at[is_v.at[0]], rb0, sem0)
    pltpu.async_copy(tbl_h.at[is_v.at[1]], rb1, sem1)
    npair = CPT // 2

    def body(p, carry):
        c0 = 2 * p
        pltpu.make_async_copy(tbl_h.at[is_v.at[c0]], rb0, sem0).wait()
        pltpu.sync_copy(rb0, acc.at[id_v.at[c0]], add=True)

        @pl.when(p + 1 < npair)
        def _():
            pltpu.async_copy(tbl_h.at[is_v.at[c0 + 2]], rb0, sem0)

        pltpu.make_async_copy(tbl_h.at[is_v.at[c0 + 1]], rb1, sem1).wait()
        pltpu.sync_copy(rb1, acc.at[id_v.at[c0 + 1]], add=True)

        @pl.when(p + 1 < npair)
        def _():
            pltpu.async_copy(tbl_h.at[is_v.at[c0 + 3]], rb1, sem1)

        return carry

    lax.fori_loop(0, npair, body, 0)


def _make_sc_scatter(fs):
    n = len(fs)
    widths = sorted(set(fs))
    scratch = []
    for f in fs:
        scratch += [
            pltpu.VMEM((CPT, CH), jnp.int32),
            pltpu.VMEM((CPT, CH), jnp.int32),
            pltpu.VMEM((CH, f), jnp.float32),
            pltpu.VMEM((CH, f), jnp.float32),
        ]
    for f in widths:
        scratch.append(pltpu.VMEM_SHARED((NPAD, f), jnp.float32))
    scratch += [pltpu.SemaphoreType.DMA, pltpu.SemaphoreType.DMA]

    def body(*refs):
        ins = refs[: 4 * n]
        outs = refs[4 * n: 5 * n]
        scr = refs[5 * n:]
        sem0, sem1 = scr[-2], scr[-1]
        accs = {f: scr[4 * n + j] for j, f in enumerate(widths)}
        core = lax.axis_index("c")
        sid = lax.axis_index("s")
        wid = core * 16 + sid
        r0 = sid * RPT
        seen = set()
        for i in range(n):
            f = fs[i]
            if f in seen:
                continue
            seen.add(f)
            zeros_h = ins[4 * i + 3]
            pltpu.sync_copy(zeros_h.at[pl.ds(r0, RPT)],
                            accs[f].at[pl.ds(r0, RPT)])
        plsc.subcore_barrier()
        for i in range(n):
            src_h, dst_h, tbl_h, zeros_h = ins[4 * i: 4 * i + 4]
            is_v, id_v, rb0, rb1 = scr[4 * i: 4 * i + 4]
            acc = accs[fs[i]]
            _edge_pass(src_h, dst_h, tbl_h, acc, is_v, id_v, rb0, rb1,
                       sem0, sem1, wid)
            plsc.subcore_barrier()
            pltpu.sync_copy(acc.at[pl.ds(r0, RPT)],
                            outs[i].at[core, pl.ds(r0, RPT)])
            if fs[i] in [fj for fj in fs[i + 1:]]:
                pltpu.sync_copy(zeros_h.at[pl.ds(r0, RPT)],
                                acc.at[pl.ds(r0, RPT)])
            if i + 1 < n:
                plsc.subcore_barrier()

    return pl.kernel(
        body,
        out_type=tuple(_f32((2, NPAD, f)) for f in fs),
        mesh=_MESH,
        scratch_types=scratch,
        compiler_params=pltpu.CompilerParams(use_tc_tiling_on_sc=False),
    )


_sc_pass_64 = _make_sc_scatter((HID,))
_sc_pass_40 = _make_sc_scatter((C,))


def _row_spec(f):
    return pl.BlockSpec((BLK, f), lambda i: (i, 0))


def _part_spec(f):
    return pl.BlockSpec((2, BLK, f), lambda i: (0, i, 0))


def _full_spec(shape):
    return pl.BlockSpec(shape, lambda i: tuple(0 for _ in shape))


def _tc1_body(x_ref, wg_ref, wh_ref, deg_ref, d_ref, b_ref,
              y1_ref, xh1_ref, dinv_ref, dinvh_ref, binv_ref):
    x = x_ref[...]
    deg = deg_ref[0, :, 0:1] + deg_ref[1, :, 0:1] + 1.0
    dinv = lax.rsqrt(deg)
    dd = d_ref[0, :, 0:1] + d_ref[1, :, 0:1]
    bb = b_ref[0, :, 0:1] + b_ref[1, :, 0:1]
    y1_ref[...] = dinv * jnp.dot(x, wg_ref[...], preferred_element_type=jnp.float32)
    xh1_ref[...] = jnp.dot(x, wh_ref[...], preferred_element_type=jnp.float32)
    dinv_ref[...] = dinv
    dinvh_ref[...] = jnp.where(dd > 0, 1.0 / dd, 0.0)
    binv_ref[...] = jnp.where(bb > 0, 1.0 / bb, 0.0)


def _tc1(x_pad, wg1, wh1, deg_p, d_p, b_p):
    return pl.pallas_call(
        _tc1_body,
        grid=(GRID,),
        in_specs=[_row_spec(F_IN), _full_spec((F_IN, HID)), _full_spec((F_IN, HID)),
                  _part_spec(HW), _part_spec(HW), _part_spec(HW)],
        out_specs=[_row_spec(HID), _row_spec(HID), _row_spec(1), _row_spec(1), _row_spec(1)],
        out_shape=[_f32((NPAD, HID)), _f32((NPAD, HID)), _f32((NPAD, 1)),
                   _f32((NPAD, 1)), _f32((NPAD, 1))],
    )(x_pad, wg1, wh1, deg_p, d_p, b_p)


def _tc2_body(s1_ref, t1_ref, y1_ref, dinv_ref, binv_ref, bg1_ref, wg2_ref,
              y2_ref, ef1_ref):
    dinv = dinv_ref[...]
    xg1 = jnp.maximum(dinv * (s1_ref[0] + s1_ref[1] + y1_ref[...]) + bg1_ref[...], 0.0)
    y2_ref[...] = dinv * jnp.dot(xg1, wg2_ref[...], preferred_element_type=jnp.float32)
    ef1_ref[...] = binv_ref[...] * (t1_ref[0] + t1_ref[1])


def _tc2(s1_p, t1_p, y1, dinv, binv, bg1, wg2):
    return pl.pallas_call(
        _tc2_body,
        grid=(GRID,),
        in_specs=[_part_spec(HID), _part_spec(HID), _row_spec(HID), _row_spec(1),
                  _row_spec(1), _full_spec((1, HID)), _full_spec((HID, C))],
        out_specs=[_row_spec(C), _row_spec(HID)],
        out_shape=[_f32((NPAD, C)), _f32((NPAD, HID))],
    )(s1_p, t1_p, y1, dinv, binv, bg1, wg2)


def _tc3_body(u1_ref, s2_ref, y2_ref, dinv_ref, dinvh_ref, bh1_ref, wh2_ref,
              bg2_ref, xh2_ref, xg2_ref):
    xhyp1 = jnp.maximum(dinvh_ref[...] * (u1_ref[0] + u1_ref[1]) + bh1_ref[...], 0.0)
    xh2_ref[...] = jnp.dot(xhyp1, wh2_ref[...], preferred_element_type=jnp.float32)
    xg2_ref[...] = dinv_ref[...] * (s2_ref[0] + s2_ref[1] + y2_ref[...]) + bg2_ref[...]


def _tc3(u1_p, s2_p, y2, dinv, dinvh, bh1, wh2, bg2):
    return pl.pallas_call(
        _tc3_body,
        grid=(GRID,),
        in_specs=[_part_spec(HID), _part_spec(C), _row_spec(C), _row_spec(1),
                  _row_spec(1), _full_spec((1, HID)), _full_spec((HID, C)),
                  _full_spec((1, C))],
        out_specs=[_row_spec(C), _row_spec(C)],
        out_shape=[_f32((NPAD, C)), _f32((NPAD, C))],
    )(u1_p, s2_p, y2, dinv, dinvh, bh1, wh2, bg2)


def _tc4_body(t2_ref, binv_ref, ef2_ref):
    ef2_ref[...] = binv_ref[...] * (t2_ref[0] + t2_ref[1])


def _tc4(t2_p, binv):
    return pl.pallas_call(
        _tc4_body,
        grid=(GRID,),
        in_specs=[_part_spec(C), _row_spec(1)],
        out_specs=[_row_spec(C)],
        out_shape=[_f32((NPAD, C))],
    )(t2_p, binv)[0]


def _tc5_body(u2_ref, dinvh_ref, bh2_ref, xg2_ref, wlp_ref, blp_ref, out_ref):
    xhyp2 = dinvh_ref[...] * (u2_ref[0] + u2_ref[1]) + bh2_ref[...]
    cat = jnp.concatenate([xg2_ref[...], xhyp2], axis=1)
    o = jnp.dot(cat, wlp_ref[...], preferred_element_type=jnp.float32) + blp_ref[...]
    m = jnp.max(o, axis=1, keepdims=True)
    s = o - m
    out_ref[...] = s - jnp.log(jnp.sum(jnp.exp(s), axis=1, keepdims=True))


def _tc5(u2_p, dinvh, bh2, xg2, wlp, blp):
    return pl.pallas_call(
        _tc5_body,
        grid=(GRID,),
        in_specs=[_part_spec(C), _row_spec(1), _full_spec((1, C)), _row_spec(C),
                  _full_spec((2 * C, C)), _full_spec((1, C))],
        out_specs=[_row_spec(C)],
        out_shape=[_f32((NPAD, C))],
    )(u2_p, dinvh, bh2, xg2, wlp, blp)[0]


def kernel(x, edge_index, hyperedge_index, W_gcn1, b_gcn1, W_hyp1, b_hyp1,
           W_gcn2, b_gcn2, W_hyp2, b_hyp2, W_lp, b_lp):
    f32 = jnp.float32
    pad = NNODE + (jnp.arange(EP - NEDGE, dtype=jnp.int32) % (NPAD - NNODE))

    def prep(a):
        return jnp.concatenate([a.astype(jnp.int32), pad]).reshape(NTILES * CPT, CH)

    row_p = prep(edge_index[0])
    col_p = prep(edge_index[1])
    nid_p = prep(hyperedge_index[0])
    hid_p = prep(hyperedge_index[1])
    x_pad = jnp.zeros((NPAD, F_IN), f32).at[:NNODE].set(x)
    ones8 = jnp.ones((CH, HW), f32)
    zeros8 = jnp.zeros((NPAD, HW), f32)
    z64 = jnp.zeros((NPAD, HID), f32)
    z40 = jnp.zeros((NPAD, C), f32)
    bg1 = b_gcn1.reshape(1, HID)
    bh1 = b_hyp1.reshape(1, HID)
    bg2 = b_gcn2.reshape(1, C)
    bh2 = b_hyp2.reshape(1, C)
    blp = b_lp.reshape(1, C)

    deg_p, d_p, b_p = _sc_hist(col_p, nid_p, hid_p, ones8, zeros8)
    y1, xh1, dinv, dinvh, binv = _tc1(x_pad, W_gcn1, W_hyp1, deg_p, d_p, b_p)
    (s1_p,) = _sc_pass_64(row_p, col_p, y1, z64)
    (t1_p,) = _sc_pass_64(nid_p, hid_p, xh1, z64)
    y2, ef1 = _tc2(s1_p, t1_p, y1, dinv, binv, bg1, W_gcn2)
    (u1_p,) = _sc_pass_64(hid_p, nid_p, ef1, z64)
    (s2_p,) = _sc_pass_40(row_p, col_p, y2, z40)
    xh2, xg2 = _tc3(u1_p, s2_p, y2, dinv, dinvh, bh1, W_hyp2, bg2)
    (t2_p,) = _sc_pass_40(nid_p, hid_p, xh2, z40)
    ef2 = _tc4(t2_p, binv)
    (u2_p,) = _sc_pass_40(hid_p, nid_p, ef2, z40)
    out = _tc5(u2_p, dinvh, bh2, xg2, W_lp, blp)
    return out[:NNODE]
